# baseline (speedup 1.0000x reference)
import jax
import jax.numpy as jnp
from jax.experimental import pallas as pl
from jax.experimental.pallas import tpu as pltpu

_TN = 512
_UNROLL = 8


def _round_up(x, m):
    return (x + m - 1) // m * m


def _issue_rolled(ids_ref, w_hbm, rows, sem, blk):
    tn = rows.shape[0]
    base = blk * tn

    @pl.loop(0, tn // _UNROLL)
    def _(r0):
        r = r0 * _UNROLL
        for u in range(_UNROLL):
            tok = ids_ref[base + r + u]
            pltpu.make_async_copy(
                w_hbm.at[tok],
                rows.at[r + u],
                sem,
            ).start()


def _issue_unrolled(ids_ref, w_hbm, rows, sem, blk):
    tn = rows.shape[0]
    base = blk * tn
    for r in range(tn):
        tok = ids_ref[base + r]
        pltpu.make_async_copy(
            w_hbm.at[tok],
            rows.at[r],
            sem,
        ).start(priority=r % 2)


def _wait_compute_store(tt_ref, tt_w_ref, o_ref, rows, sem):
    pltpu.make_async_copy(rows, rows, sem).wait()
    x = rows[...].astype(jnp.float32)
    tt = tt_ref[...]
    T = tt_w_ref.shape[0]
    if T == 2:
        sel = jnp.where(tt == 0,
                        tt_w_ref[0].astype(jnp.float32),
                        tt_w_ref[1].astype(jnp.float32))
        x = x + sel
    else:
        for t in range(T):
            x = x + jnp.where(tt == t, tt_w_ref[t].astype(jnp.float32), 0.0)
    o_ref[...] = x.astype(o_ref.dtype)


def _gather_tt_kernel(ids_ref, tt_ref, w_hbm, tt_w_ref, o_ref,
                      rows0, rows1, sems):
    i = pl.program_id(0)
    nblk = pl.num_programs(0)
    even = i % 2 == 0

    @pl.when(i == 0)
    def _():
        _issue_rolled(ids_ref, w_hbm, rows0, sems.at[0], 0)

    @pl.when(jnp.logical_and(even, i + 1 < nblk))
    def _():
        _issue_unrolled(ids_ref, w_hbm, rows1, sems.at[1], i + 1)

    @pl.when(jnp.logical_and(~even, i + 1 < nblk))
    def _():
        _issue_unrolled(ids_ref, w_hbm, rows0, sems.at[0], i + 1)

    @pl.when(even)
    def _():
        _wait_compute_store(tt_ref, tt_w_ref, o_ref, rows0, sems.at[0])

    @pl.when(~even)
    def _():
        _wait_compute_store(tt_ref, tt_w_ref, o_ref, rows1, sems.at[1])


@jax.jit
def _embed(input_ids, word_weight, tokentype_ids, tokentype_weight):
    B, S = input_ids.shape
    V, H = word_weight.shape
    out_dtype = word_weight.dtype
    N = B * S

    H_pad = _round_up(H, 1024)
    if H_pad != H:
        word_weight = jnp.pad(word_weight, ((0, 0), (0, H_pad - H)))
        tokentype_weight = jnp.pad(tokentype_weight,
                                   ((0, 0), (0, H_pad - H)))
    S8 = H_pad // 128
    T = tokentype_weight.shape[0]

    tn = min(_TN, _round_up(N, 8))
    N_pad = _round_up(N, tn)
    nblk = N_pad // tn

    ids_flat = input_ids.reshape(N).astype(jnp.int32)
    tt_flat = tokentype_ids.reshape(N).astype(jnp.int32)
    if N_pad != N:
        ids_flat = jnp.pad(ids_flat, (0, N_pad - N))
        tt_flat = jnp.pad(tt_flat, (0, N_pad - N))
    tt3 = tt_flat.reshape(N_pad, 1, 1)

    w3 = word_weight.reshape(V, S8, 128)
    ttw3 = tokentype_weight.reshape(T, S8, 128)

    out = pl.pallas_call(
        _gather_tt_kernel,
        out_shape=jax.ShapeDtypeStruct((N_pad, S8, 128), out_dtype),
        grid_spec=pltpu.PrefetchScalarGridSpec(
            num_scalar_prefetch=1,
            grid=(nblk,),
            in_specs=[
                pl.BlockSpec((tn, 1, 1), lambda i, ids: (i, 0, 0)),
                pl.BlockSpec(memory_space=pl.ANY),
                pl.BlockSpec((T, S8, 128), lambda i, ids: (0, 0, 0)),
            ],
            out_specs=pl.BlockSpec((tn, S8, 128), lambda i, ids: (i, 0, 0)),
            scratch_shapes=[pltpu.VMEM((tn, S8, 128), word_weight.dtype),
                            pltpu.VMEM((tn, S8, 128), word_weight.dtype),
                            pltpu.SemaphoreType.DMA((2,))],
        ),
        compiler_params=pltpu.CompilerParams(
            dimension_semantics=("arbitrary",),
            disable_bounds_checks=True,
        ),
    )(ids_flat, tt3, w3, ttw3)

    out = out.reshape(N_pad, H_pad)[:N, :H]
    return out.reshape(B, S, H)


def kernel(input_ids, word_weight, tokentype_ids, tokentype_weight):
    return _embed(input_ids, word_weight, tokentype_ids, tokentype_weight)


# speedup vs baseline: 3.7335x; 3.7335x over previous
import jax
import jax.numpy as jnp
from jax.experimental import pallas as pl
from jax.experimental.pallas import tpu as pltpu

_TN = 512
_UNROLL = 8


def _round_up(x, m):
    return (x + m - 1) // m * m


def _issue_block_rolled(ids_ref, w_hbm, rows, sems, blk, slot):
    tn = rows.shape[1]
    base = blk * tn

    @pl.loop(0, tn // _UNROLL)
    def _(r0):
        r = r0 * _UNROLL
        for u in range(_UNROLL):
            tok = ids_ref[base + r + u]
            pltpu.make_async_copy(
                w_hbm.at[pl.ds(tok, 1), :],
                rows.at[slot, pl.ds(r + u, 1), :],
                sems.at[slot],
            ).start()


def _issue_block_unrolled(ids_ref, w_hbm, rows, sems, blk, slot):
    tn = rows.shape[1]
    base = blk * tn
    for r in range(tn):
        tok = ids_ref[base + r]
        pltpu.make_async_copy(
            w_hbm.at[pl.ds(tok, 1), :],
            rows.at[slot, pl.ds(r, 1), :],
            sems.at[slot],
        ).start(priority=r % 2)


def _wait_compute_store(tt_ref, tt_w_ref, o_ref, rows, sems, slot):
    pltpu.make_async_copy(rows.at[slot], rows.at[slot], sems.at[slot]).wait()
    x = rows[slot].astype(jnp.float32)
    tt = tt_ref[...]
    T = tt_w_ref.shape[0]
    if T == 2:
        sel = jnp.where(tt == 0,
                        tt_w_ref[0:1, :].astype(jnp.float32),
                        tt_w_ref[1:2, :].astype(jnp.float32))
        x = x + sel
    else:
        for t in range(T):
            row_t = tt_w_ref[pl.ds(t, 1), :].astype(jnp.float32)
            x = x + jnp.where(tt == t, row_t, 0.0)
    o_ref[...] = x.astype(o_ref.dtype)


def _gather_tt_kernel(ids_ref, tt_ref, w_hbm, tt_w_ref, o_ref, rows, sems):
    i = pl.program_id(0)
    nblk = pl.num_programs(0)
    even = i % 2 == 0

    @pl.when(i == 0)
    def _():
        _issue_block_rolled(ids_ref, w_hbm, rows, sems, 0, 0)

    @pl.when(jnp.logical_and(even, i + 1 < nblk))
    def _():
        _issue_block_unrolled(ids_ref, w_hbm, rows, sems, i + 1, 1)

    @pl.when(jnp.logical_and(~even, i + 1 < nblk))
    def _():
        _issue_block_unrolled(ids_ref, w_hbm, rows, sems, i + 1, 0)

    @pl.when(even)
    def _():
        _wait_compute_store(tt_ref, tt_w_ref, o_ref, rows, sems, 0)

    @pl.when(~even)
    def _():
        _wait_compute_store(tt_ref, tt_w_ref, o_ref, rows, sems, 1)


@jax.jit
def _embed(input_ids, word_weight, tokentype_ids, tokentype_weight):
    B, S = input_ids.shape
    V, H = word_weight.shape
    out_dtype = word_weight.dtype
    N = B * S

    H_pad = _round_up(H, 128)
    if H_pad != H:
        word_weight = jnp.pad(word_weight, ((0, 0), (0, H_pad - H)))
        tokentype_weight = jnp.pad(tokentype_weight,
                                   ((0, 0), (0, H_pad - H)))

    tn = min(_TN, _round_up(N, 8))
    N_pad = _round_up(N, tn)
    nblk = N_pad // tn
    T = tokentype_weight.shape[0]

    ids_flat = input_ids.reshape(N).astype(jnp.int32)
    tt_flat = tokentype_ids.reshape(N).astype(jnp.int32)
    if N_pad != N:
        ids_flat = jnp.pad(ids_flat, (0, N_pad - N))
        tt_flat = jnp.pad(tt_flat, (0, N_pad - N))
    tt_flat = tt_flat.reshape(N_pad, 1)

    out = pl.pallas_call(
        _gather_tt_kernel,
        out_shape=jax.ShapeDtypeStruct((N_pad, H_pad), out_dtype),
        grid_spec=pltpu.PrefetchScalarGridSpec(
            num_scalar_prefetch=1,
            grid=(nblk,),
            in_specs=[
                pl.BlockSpec((tn, 1), lambda i, ids: (i, 0)),
                pl.BlockSpec(memory_space=pl.ANY),
                pl.BlockSpec((T, H_pad), lambda i, ids: (0, 0)),
            ],
            out_specs=pl.BlockSpec((tn, H_pad), lambda i, ids: (i, 0)),
            scratch_shapes=[pltpu.VMEM((2, tn, H_pad), word_weight.dtype),
                            pltpu.SemaphoreType.DMA((2,))],
        ),
        compiler_params=pltpu.CompilerParams(
            dimension_semantics=("arbitrary",),
            disable_bounds_checks=True,
        ),
    )(ids_flat, tt_flat, word_weight, tokentype_weight)

    return out[:N, :H].reshape(B, S, H)


def kernel(input_ids, word_weight, tokentype_ids, tokentype_weight):
    return _embed(input_ids, word_weight, tokentype_ids, tokentype_weight)


# speedup vs baseline: 3.7376x; 1.0011x over previous
import jax
import jax.numpy as jnp
from jax.experimental import pallas as pl
from jax.experimental.pallas import tpu as pltpu

_TN = 1024
_UNROLL = 8


def _round_up(x, m):
    return (x + m - 1) // m * m


def _issue_block_rolled(ids_ref, w_hbm, rows, sems, blk, slot):
    tn = rows.shape[1]
    base = blk * tn

    @pl.loop(0, tn // _UNROLL)
    def _(r0):
        r = r0 * _UNROLL
        for u in range(_UNROLL):
            tok = ids_ref[base + r + u]
            pltpu.make_async_copy(
                w_hbm.at[pl.ds(tok, 1), :],
                rows.at[slot, pl.ds(r + u, 1), :],
                sems.at[slot],
            ).start()


def _issue_block_unrolled(ids_ref, w_hbm, rows, sems, blk, slot):
    tn = rows.shape[1]
    base = blk * tn
    for r in range(tn):
        tok = ids_ref[base + r]
        pltpu.make_async_copy(
            w_hbm.at[pl.ds(tok, 1), :],
            rows.at[slot, pl.ds(r, 1), :],
            sems.at[slot],
        ).start(priority=r % 2)


def _wait_compute_store(tt_ref, tt_w_ref, o_ref, rows, sems, slot):
    pltpu.make_async_copy(rows.at[slot], rows.at[slot], sems.at[slot]).wait()
    x = rows[slot].astype(jnp.float32)
    tt = tt_ref[...]
    T = tt_w_ref.shape[0]
    if T == 2:
        sel = jnp.where(tt == 0,
                        tt_w_ref[0:1, :].astype(jnp.float32),
                        tt_w_ref[1:2, :].astype(jnp.float32))
        x = x + sel
    else:
        for t in range(T):
            row_t = tt_w_ref[pl.ds(t, 1), :].astype(jnp.float32)
            x = x + jnp.where(tt == t, row_t, 0.0)
    o_ref[...] = x.astype(o_ref.dtype)


def _gather_tt_kernel(ids_ref, tt_ref, w_hbm, tt_w_ref, o_ref, rows, sems):
    i = pl.program_id(0)
    nblk = pl.num_programs(0)
    even = i % 2 == 0

    @pl.when(i == 0)
    def _():
        _issue_block_rolled(ids_ref, w_hbm, rows, sems, 0, 0)

    @pl.when(jnp.logical_and(even, i + 1 < nblk))
    def _():
        _issue_block_unrolled(ids_ref, w_hbm, rows, sems, i + 1, 1)

    @pl.when(jnp.logical_and(~even, i + 1 < nblk))
    def _():
        _issue_block_unrolled(ids_ref, w_hbm, rows, sems, i + 1, 0)

    @pl.when(even)
    def _():
        _wait_compute_store(tt_ref, tt_w_ref, o_ref, rows, sems, 0)

    @pl.when(~even)
    def _():
        _wait_compute_store(tt_ref, tt_w_ref, o_ref, rows, sems, 1)


@jax.jit
def _embed(input_ids, word_weight, tokentype_ids, tokentype_weight):
    B, S = input_ids.shape
    V, H = word_weight.shape
    out_dtype = word_weight.dtype
    N = B * S

    H_pad = _round_up(H, 128)
    if H_pad != H:
        word_weight = jnp.pad(word_weight, ((0, 0), (0, H_pad - H)))
        tokentype_weight = jnp.pad(tokentype_weight,
                                   ((0, 0), (0, H_pad - H)))

    tn = min(_TN, _round_up(N, 8))
    N_pad = _round_up(N, tn)
    nblk = N_pad // tn
    T = tokentype_weight.shape[0]

    ids_flat = input_ids.reshape(N).astype(jnp.int32)
    tt_flat = tokentype_ids.reshape(N).astype(jnp.int32)
    if N_pad != N:
        ids_flat = jnp.pad(ids_flat, (0, N_pad - N))
        tt_flat = jnp.pad(tt_flat, (0, N_pad - N))
    tt_flat = tt_flat.reshape(N_pad, 1)

    out = pl.pallas_call(
        _gather_tt_kernel,
        out_shape=jax.ShapeDtypeStruct((N_pad, H_pad), out_dtype),
        grid_spec=pltpu.PrefetchScalarGridSpec(
            num_scalar_prefetch=1,
            grid=(nblk,),
            in_specs=[
                pl.BlockSpec((tn, 1), lambda i, ids: (i, 0)),
                pl.BlockSpec(memory_space=pl.ANY),
                pl.BlockSpec((T, H_pad), lambda i, ids: (0, 0)),
            ],
            out_specs=pl.BlockSpec((tn, H_pad), lambda i, ids: (i, 0)),
            scratch_shapes=[pltpu.VMEM((2, tn, H_pad), word_weight.dtype),
                            pltpu.SemaphoreType.DMA((2,))],
        ),
        compiler_params=pltpu.CompilerParams(
            dimension_semantics=("arbitrary",),
            disable_bounds_checks=True,
        ),
    )(ids_flat, tt_flat, word_weight, tokentype_weight)

    return out[:N, :H].reshape(B, S, H)


def kernel(input_ids, word_weight, tokentype_ids, tokentype_weight):
    return _embed(input_ids, word_weight, tokentype_ids, tokentype_weight)


# speedup vs baseline: 3.7924x; 1.0147x over previous
import jax
import jax.numpy as jnp
from jax.experimental import pallas as pl
from jax.experimental.pallas import tpu as pltpu

_TN = 512


def _round_up(x, m):
    return (x + m - 1) // m * m


def _issue_block_unrolled(ids_ref, w_hbm, rows, sems, blk, slot):
    tn = rows.shape[1]
    base = blk * tn
    for r in range(tn):
        tok = ids_ref[base + r]
        pltpu.make_async_copy(
            w_hbm.at[pl.ds(tok, 1), :],
            rows.at[slot, pl.ds(r, 1), :],
            sems.at[slot],
        ).start(priority=r % 2)


def _wait_compute_store(tt_ref, tt_w_ref, o_ref, rows, sems, slot):
    pltpu.make_async_copy(rows.at[slot], rows.at[slot], sems.at[slot]).wait()
    x = rows[slot].astype(jnp.float32)
    tt = tt_ref[...]
    T = tt_w_ref.shape[0]
    if T == 2:
        sel = jnp.where(tt == 0,
                        tt_w_ref[0:1, :].astype(jnp.float32),
                        tt_w_ref[1:2, :].astype(jnp.float32))
        x = x + sel
    else:
        for t in range(T):
            row_t = tt_w_ref[pl.ds(t, 1), :].astype(jnp.float32)
            x = x + jnp.where(tt == t, row_t, 0.0)
    o_ref[...] = x.astype(o_ref.dtype)


def _gather_tt_kernel(ids_ref, tt_ref, w_hbm, tt_w_ref, o_ref, rows, sems):
    i = pl.program_id(0)
    nblk = pl.num_programs(0)
    even = i % 2 == 0

    @pl.when(i == 0)
    def _():
        _issue_block_unrolled(ids_ref, w_hbm, rows, sems, 0, 0)

    @pl.when(jnp.logical_and(even, i + 1 < nblk))
    def _():
        _issue_block_unrolled(ids_ref, w_hbm, rows, sems, i + 1, 1)

    @pl.when(jnp.logical_and(~even, i + 1 < nblk))
    def _():
        _issue_block_unrolled(ids_ref, w_hbm, rows, sems, i + 1, 0)

    @pl.when(even)
    def _():
        _wait_compute_store(tt_ref, tt_w_ref, o_ref, rows, sems, 0)

    @pl.when(~even)
    def _():
        _wait_compute_store(tt_ref, tt_w_ref, o_ref, rows, sems, 1)


@jax.jit
def _embed(input_ids, word_weight, tokentype_ids, tokentype_weight):
    B, S = input_ids.shape
    V, H = word_weight.shape
    out_dtype = word_weight.dtype
    N = B * S

    H_pad = _round_up(H, 128)
    if H_pad != H:
        word_weight = jnp.pad(word_weight, ((0, 0), (0, H_pad - H)))
        tokentype_weight = jnp.pad(tokentype_weight,
                                   ((0, 0), (0, H_pad - H)))

    tn = min(_TN, _round_up(N, 8))
    N_pad = _round_up(N, tn)
    nblk = N_pad // tn
    T = tokentype_weight.shape[0]

    ids_flat = input_ids.reshape(N).astype(jnp.int32)
    tt_flat = tokentype_ids.reshape(N).astype(jnp.int32)
    if N_pad != N:
        ids_flat = jnp.pad(ids_flat, (0, N_pad - N))
        tt_flat = jnp.pad(tt_flat, (0, N_pad - N))
    tt_flat = tt_flat.reshape(N_pad, 1)

    out = pl.pallas_call(
        _gather_tt_kernel,
        out_shape=jax.ShapeDtypeStruct((N_pad, H_pad), out_dtype),
        grid_spec=pltpu.PrefetchScalarGridSpec(
            num_scalar_prefetch=1,
            grid=(nblk,),
            in_specs=[
                pl.BlockSpec((tn, 1), lambda i, ids: (i, 0)),
                pl.BlockSpec(memory_space=pl.ANY),
                pl.BlockSpec((T, H_pad), lambda i, ids: (0, 0)),
            ],
            out_specs=pl.BlockSpec((tn, H_pad), lambda i, ids: (i, 0)),
            scratch_shapes=[pltpu.VMEM((2, tn, H_pad), word_weight.dtype),
                            pltpu.SemaphoreType.DMA((2,))],
        ),
        compiler_params=pltpu.CompilerParams(
            dimension_semantics=("arbitrary",),
            disable_bounds_checks=True,
        ),
    )(ids_flat, tt_flat, word_weight, tokentype_weight)

    return out[:N, :H].reshape(B, S, H)


def kernel(input_ids, word_weight, tokentype_ids, tokentype_weight):
    return _embed(input_ids, word_weight, tokentype_ids, tokentype_weight)
